# baseline (speedup 1.0000x reference)
import jax
import jax.numpy as jnp
from jax.experimental import pallas as pl

_B, _SEG, _D, _H, _O = 16, 2048, 1024, 256, 32


_SPB = 2


def _readout_body(x_ref, w1_ref, w2_ref, o_ref):
    w2 = w2_ref[...]
    w1b = w1_ref[...].astype(jnp.bfloat16)
    w2b = w2.astype(jnp.bfloat16)
    k = jnp.sum(jnp.abs(w2), axis=1)
    for seg in range(_SPB):
        x = x_ref[seg * _SEG:(seg + 1) * _SEG, :]
        xb = x.astype(jnp.bfloat16)
        t = jnp.tanh(jnp.dot(xb, w1b.T, preferred_element_type=jnp.float32))
        s = jnp.dot(t.astype(jnp.bfloat16), w2b.T,
                    preferred_element_type=jnp.float32)
        e = jnp.exp(s - k[None, :])
        l = jnp.sum(e, axis=0)
        acc = jax.lax.dot_general(
            e.astype(jnp.bfloat16), xb, (((0,), (0,)), ((), ())),
            preferred_element_type=jnp.float32)
        o_ref[seg * _O:(seg + 1) * _O, :] = acc / l[:, None]


def kernel(embeddings, scope, w1, w2):
    del scope
    out = pl.pallas_call(
        _readout_body,
        grid=(_B // _SPB,),
        in_specs=[
            pl.BlockSpec((_SPB * _SEG, _D), lambda b: (b, 0)),
            pl.BlockSpec((_H, _D), lambda b: (0, 0)),
            pl.BlockSpec((_O, _H), lambda b: (0, 0)),
        ],
        out_specs=pl.BlockSpec((_SPB * _O, _D), lambda b: (b, 0)),
        out_shape=jax.ShapeDtypeStruct((_B * _O, _D), jnp.float32),
    )(embeddings, w1, w2)
    return out.reshape(_B, _O * _D)
